# baseline (speedup 1.0000x reference)
import jax
import jax.numpy as jnp
from jax.experimental import pallas as pl
from jax.experimental.pallas import tpu as pltpu

_IB = 8


def _fused_kernel(conf_ref, tgt_ref, locp_ref, loct_ref, out_ref, acc_ref):
    b = pl.program_id(0)
    nb = pl.num_programs(0)
    c = conf_ref.shape[0]
    n = conf_ref.shape[2]
    y = tgt_ref[...]
    x0 = conf_ref[0]
    s = jnp.exp(x0)
    xy = jnp.where(y == 0, x0, 0.0)
    for ci in range(1, c):
        xc = conf_ref[ci]
        s = s + jnp.exp(xc)
        xy = jnp.where(y == ci, xc, xy)
    ce = jnp.log(s) - xy

    posf = (y > 0).astype(jnp.float32)
    d = locp_ref[...] - loct_ref[...]
    ad = jnp.abs(d)
    sl1 = jnp.where(ad < 1.0, 0.5 * d * d, ad - 0.5)
    lsum = jnp.sum(jnp.sum(sl1, axis=1) * posf)

    num_pos = jnp.sum(posf, axis=1, keepdims=True)
    ce_pos_sum = jnp.sum(ce * posf)
    cl = ce * (1.0 - posf)
    bits = jax.lax.bitcast_convert_type(cl, jnp.int32)
    k = jnp.clip(3 * num_pos.astype(jnp.int32), 1, n - 1)

    hi0 = jnp.max(bits, axis=1, keepdims=True)
    lo0 = jnp.zeros_like(hi0)

    def body(_, carry):
        lo, hi = carry
        mid = lo + ((hi - lo + 1) >> 1)
        cnt = jnp.sum((bits >= mid).astype(jnp.int32), axis=1, keepdims=True)
        pred = cnt >= k
        return jnp.where(pred, mid, lo), jnp.where(pred, hi, mid - 1)

    tbits, _ = jax.lax.fori_loop(0, 31, body, (lo0, hi0))
    t = jax.lax.bitcast_convert_type(tbits, jnp.float32)
    gt = bits > tbits
    g = jnp.sum(gt.astype(jnp.float32), axis=1, keepdims=True)
    sgt = jnp.sum(jnp.where(gt, cl, 0.0), axis=1, keepdims=True)
    sneg_sum = jnp.sum(sgt + (k.astype(jnp.float32) - g) * t)

    loss_part = lsum + ce_pos_sum + sneg_sum
    np_part = jnp.sum(num_pos)
    lane = jax.lax.broadcasted_iota(jnp.int32, (1, 128), 1)
    vec = jnp.where(lane == 0, loss_part, jnp.where(lane == 1, np_part, 0.0))

    @pl.when(b == 0)
    def _():
        acc_ref[...] = jnp.zeros_like(acc_ref)

    acc_ref[...] = acc_ref[...] + vec

    @pl.when(b == nb - 1)
    def _():
        out_ref[...] = jnp.full((1, 1), acc_ref[0, 0] / acc_ref[0, 1])


def kernel(loc_preds, conf_preds, loc_targets, conf_targets):
    B, N, C = conf_preds.shape
    conf_t = jnp.transpose(conf_preds, (2, 0, 1))
    locp_t = jnp.transpose(loc_preds, (0, 2, 1))
    loct_t = jnp.transpose(loc_targets, (0, 2, 1))
    tgt = conf_targets.astype(jnp.int32)

    out = pl.pallas_call(
        _fused_kernel,
        grid=(B // _IB,),
        in_specs=[
            pl.BlockSpec((C, _IB, N), lambda b: (0, b, 0)),
            pl.BlockSpec((_IB, N), lambda b: (b, 0)),
            pl.BlockSpec((_IB, 4, N), lambda b: (b, 0, 0)),
            pl.BlockSpec((_IB, 4, N), lambda b: (b, 0, 0)),
        ],
        out_specs=pl.BlockSpec((1, 1), lambda b: (0, 0)),
        out_shape=jax.ShapeDtypeStruct((1, 1), jnp.float32),
        scratch_shapes=[pltpu.VMEM((1, 128), jnp.float32)],
    )(conf_t, tgt, locp_t, loct_t)

    return out[0, 0]
